# baseline (speedup 1.0000x reference)
import functools

import jax
import jax.numpy as jnp
from jax import lax
from jax.experimental import pallas as pl
from jax.experimental.pallas import tpu as pltpu
from jax.experimental.pallas import tpu_sc as plsc

N = 10000
E = 320000
D = 128
G = 64

NUM_CORES = 2
NUM_SUBCORES = 16
NUM_WORKERS = NUM_CORES * NUM_SUBCORES
CHUNK = 128
NBUF = 1
STEPS = 80
ROUNDS = STEPS // NBUF
EDGES_PER_WORKER = STEPS * CHUNK
E_PAD = NUM_WORKERS * EDGES_PER_WORKER
N_ACC = 10112
STRIPE = N_ACC // NUM_SUBCORES

ROWS_BLK = 400
N_BLOCKS = N // ROWS_BLK


@functools.lru_cache(maxsize=1)
def _make_sc_agg():
    mesh = plsc.VectorSubcoreMesh(core_axis_name="c", subcore_axis_name="s",
                                  num_cores=NUM_CORES, num_subcores=NUM_SUBCORES)

    @functools.partial(
        pl.kernel,
        mesh=mesh,
        out_type=jax.ShapeDtypeStruct((NUM_CORES, N_ACC, D), jnp.float32),
        scratch_types=[
            pltpu.VMEM((STEPS, CHUNK), jnp.int32),
            pltpu.VMEM((STEPS, CHUNK), jnp.int32),
            pltpu.VMEM((CHUNK, D), jnp.float32),
            pltpu.VMEM_SHARED((N_ACC, D), jnp.float32),
            pltpu.SemaphoreType.DMA,
        ],
    )
    def agg(feat, srcr, dstr, zeros, out, src_idx, dst_idx, rows, acc, sem):
        cid = lax.axis_index("c")
        sid = lax.axis_index("s")
        wid = sid * NUM_CORES + cid

        pltpu.sync_copy(zeros, acc.at[pl.ds(sid * STRIPE, STRIPE)])

        pltpu.sync_copy(srcr.at[wid], src_idx)
        pltpu.sync_copy(dstr.at[wid], dst_idx)
        plsc.subcore_barrier()

        def step(j, carry):
            pltpu.async_copy(feat.at[src_idx.at[j]], rows, sem).wait()
            pltpu.sync_copy(rows, acc.at[dst_idx.at[j]], add=True)
            return carry

        lax.fori_loop(0, STEPS, step, 0)
        plsc.subcore_barrier()

        pltpu.sync_copy(
            acc.at[pl.ds(sid * STRIPE, STRIPE)],
            out.at[cid, pl.ds(sid * STRIPE, STRIPE)],
        )

    return agg


def _mm_relu_body(x_ref, p_ref, w_ref, b_ref, o_ref):
    xa = x_ref[...] + p_ref[0] + p_ref[1]
    h = jnp.dot(xa, w_ref[...], preferred_element_type=jnp.float32)
    o_ref[...] = jnp.maximum(h + b_ref[...], 0.0)


def _mm_relu(x, parts, w, b):
    return pl.pallas_call(
        _mm_relu_body,
        grid=(N_BLOCKS,),
        in_specs=[
            pl.BlockSpec((ROWS_BLK, D), lambda i: (i, 0)),
            pl.BlockSpec((NUM_CORES, ROWS_BLK, D), lambda i: (0, i, 0)),
            pl.BlockSpec((D, D), lambda i: (0, 0)),
            pl.BlockSpec((1, D), lambda i: (0, 0)),
        ],
        out_specs=pl.BlockSpec((ROWS_BLK, D), lambda i: (i, 0)),
        out_shape=jax.ShapeDtypeStruct((N, D), jnp.float32),
    )(x, parts, w, b)


def _mm_pool_body(x_ref, p_ref, w_ref, b_ref, batch_ref, w3_ref, b3_ref,
                  out_ref, sums_ref, cnts_ref):
    i = pl.program_id(0)
    xa = x_ref[...] + p_ref[0] + p_ref[1]
    h = jnp.dot(xa, w_ref[...], preferred_element_type=jnp.float32)
    h = jnp.maximum(h + b_ref[...], 0.0)

    seg = batch_ref[0, 0]
    onehot = (seg[:, None] == lax.broadcasted_iota(jnp.int32, (ROWS_BLK, G), 1))
    onehot = onehot.astype(jnp.float32)
    psum = lax.dot_general(onehot, h, (((0,), (0,)), ((), ())),
                           preferred_element_type=jnp.float32)
    pcnt = lax.dot_general(onehot, jnp.ones((ROWS_BLK, D), jnp.float32),
                           (((0,), (0,)), ((), ())),
                           preferred_element_type=jnp.float32)

    @pl.when(i == 0)
    def _():
        sums_ref[...] = jnp.zeros_like(sums_ref)
        cnts_ref[...] = jnp.zeros_like(cnts_ref)

    sums_ref[...] += psum
    cnts_ref[...] += pcnt

    @pl.when(i == N_BLOCKS - 1)
    def _():
        pooled = sums_ref[...] / jnp.maximum(cnts_ref[...], 1.0)
        logits = jnp.dot(pooled, w3_ref[...], preferred_element_type=jnp.float32)
        out_ref[...] = logits + b3_ref[...]


def _mm_pool(h1, parts, w2, b2, batch_r, w3, b3):
    c = w3.shape[1]
    out, _, _ = pl.pallas_call(
        _mm_pool_body,
        grid=(N_BLOCKS,),
        in_specs=[
            pl.BlockSpec((ROWS_BLK, D), lambda i: (i, 0)),
            pl.BlockSpec((NUM_CORES, ROWS_BLK, D), lambda i: (0, i, 0)),
            pl.BlockSpec((D, D), lambda i: (0, 0)),
            pl.BlockSpec((1, D), lambda i: (0, 0)),
            pl.BlockSpec((1, 1, ROWS_BLK), lambda i: (i, 0, 0)),
            pl.BlockSpec((D, c), lambda i: (0, 0)),
            pl.BlockSpec((1, c), lambda i: (0, 0)),
        ],
        out_specs=[
            pl.BlockSpec((G, c), lambda i: (0, 0)),
            pl.BlockSpec((G, D), lambda i: (0, 0)),
            pl.BlockSpec((G, D), lambda i: (0, 0)),
        ],
        out_shape=[
            jax.ShapeDtypeStruct((G, c), jnp.float32),
            jax.ShapeDtypeStruct((G, D), jnp.float32),
            jax.ShapeDtypeStruct((G, D), jnp.float32),
        ],
    )(h1, parts, w2, b2, batch_r, w3, b3)
    return out


def kernel(x, edge_index, batch, W1, b1, W2, b2, W3, b3):
    src = edge_index[0]
    dst = edge_index[1]

    pad = E_PAD - E
    src_p = jnp.concatenate([src, jnp.zeros((pad,), jnp.int32)])
    junk = N + jnp.arange(pad, dtype=jnp.int32) % (N_ACC - N)
    dst_p = jnp.concatenate([dst, junk])
    srcr = src_p.reshape(NUM_WORKERS, STEPS, CHUNK)
    dstr = dst_p.reshape(NUM_WORKERS, STEPS, CHUNK)
    zeros = jnp.zeros((STRIPE, D), jnp.float32)

    sc_agg = _make_sc_agg()
    parts1 = sc_agg(x, srcr, dstr, zeros)
    h1 = _mm_relu(x, parts1, W1, b1.reshape(1, D))
    parts2 = sc_agg(h1, srcr, dstr, zeros)

    batch_r = batch.reshape(N_BLOCKS, 1, ROWS_BLK)
    out = _mm_pool(h1, parts2, W2, b2.reshape(1, D), batch_r,
                   W3, b3.reshape(1, -1))
    return out


# speedup vs baseline: 2.8359x; 2.8359x over previous
import functools

import jax
import jax.numpy as jnp
from jax import lax
from jax.experimental import pallas as pl
from jax.experimental.pallas import tpu as pltpu
from jax.experimental.pallas import tpu_sc as plsc

N = 10000
E = 320000
D = 128
G = 64

NUM_CORES = 2
NUM_SUBCORES = 16
NUM_WORKERS = NUM_CORES * NUM_SUBCORES
CHUNK = 128
NBUF = 1
STEPS = 80
ROUNDS = STEPS // NBUF
EDGES_PER_WORKER = STEPS * CHUNK
E_PAD = NUM_WORKERS * EDGES_PER_WORKER
N_ACC = 10112
STRIPE = N_ACC // NUM_SUBCORES

ROWS_BLK = 400
N_BLOCKS = N // ROWS_BLK


@functools.lru_cache(maxsize=1)
def _make_sc_agg():
    mesh = plsc.VectorSubcoreMesh(core_axis_name="c", subcore_axis_name="s",
                                  num_cores=NUM_CORES, num_subcores=NUM_SUBCORES)

    @functools.partial(
        pl.kernel,
        mesh=mesh,
        out_type=jax.ShapeDtypeStruct((NUM_CORES, N_ACC, D), jnp.float32),
        scratch_types=[
            pltpu.VMEM((STEPS, CHUNK), jnp.int32),
            pltpu.VMEM((STEPS, CHUNK), jnp.int32),
            pltpu.VMEM((CHUNK, D), jnp.float32),
            pltpu.VMEM_SHARED((N_ACC, D), jnp.float32),
            pltpu.SemaphoreType.DMA,
        ],
    )
    def agg(feat, srcr, dstr, zeros, out, src_idx, dst_idx, rows, acc, sem):
        cid = lax.axis_index("c")
        sid = lax.axis_index("s")
        wid = sid * NUM_CORES + cid

        pltpu.sync_copy(zeros, acc.at[pl.ds(sid * STRIPE, STRIPE)])

        pltpu.sync_copy(srcr.at[wid], src_idx)
        pltpu.sync_copy(dstr.at[wid], dst_idx)
        plsc.subcore_barrier()

        def step(j, carry):
            pltpu.async_copy(feat.at[src_idx.at[j]], rows, sem).wait()
            pltpu.sync_copy(rows, acc.at[dst_idx.at[j]], add=True)
            return carry

        lax.fori_loop(0, STEPS, step, 0)
        plsc.subcore_barrier()

        pltpu.sync_copy(
            acc.at[pl.ds(sid * STRIPE, STRIPE)],
            out.at[cid, pl.ds(sid * STRIPE, STRIPE)],
        )

    return agg


def _mm_relu_body(x_ref, p_ref, w_ref, b_ref, o_ref):
    xa = x_ref[...] + p_ref[0] + p_ref[1]
    h = jnp.dot(xa, w_ref[...], preferred_element_type=jnp.float32)
    o_ref[...] = jnp.maximum(h + b_ref[...], 0.0)


def _mm_relu(x, parts, w, b):
    return pl.pallas_call(
        _mm_relu_body,
        grid=(N_BLOCKS,),
        in_specs=[
            pl.BlockSpec((ROWS_BLK, D), lambda i: (i, 0)),
            pl.BlockSpec((NUM_CORES, ROWS_BLK, D), lambda i: (0, i, 0)),
            pl.BlockSpec((D, D), lambda i: (0, 0)),
            pl.BlockSpec((1, D), lambda i: (0, 0)),
        ],
        out_specs=pl.BlockSpec((ROWS_BLK, D), lambda i: (i, 0)),
        out_shape=jax.ShapeDtypeStruct((N, D), jnp.float32),
    )(x, parts, w, b)


def _mm_pool_body(x_ref, p_ref, w_ref, b_ref, batch_ref, w3_ref, b3_ref,
                  out_ref, sums_ref, cnts_ref):
    i = pl.program_id(0)
    xa = x_ref[...] + p_ref[0] + p_ref[1]
    h = jnp.dot(xa, w_ref[...], preferred_element_type=jnp.float32)
    h = jnp.maximum(h + b_ref[...], 0.0)

    seg = batch_ref[0, 0]
    onehot = (seg[:, None] == lax.broadcasted_iota(jnp.int32, (ROWS_BLK, G), 1))
    onehot = onehot.astype(jnp.float32)
    psum = lax.dot_general(onehot, h, (((0,), (0,)), ((), ())),
                           preferred_element_type=jnp.float32)
    pcnt = lax.dot_general(onehot, jnp.ones((ROWS_BLK, D), jnp.float32),
                           (((0,), (0,)), ((), ())),
                           preferred_element_type=jnp.float32)

    @pl.when(i == 0)
    def _():
        sums_ref[...] = jnp.zeros_like(sums_ref)
        cnts_ref[...] = jnp.zeros_like(cnts_ref)

    sums_ref[...] += psum
    cnts_ref[...] += pcnt

    @pl.when(i == N_BLOCKS - 1)
    def _():
        pooled = sums_ref[...] / jnp.maximum(cnts_ref[...], 1.0)
        logits = jnp.dot(pooled, w3_ref[...], preferred_element_type=jnp.float32)
        out_ref[...] = logits + b3_ref[...]


def _mm_pool(h1, parts, w2, b2, batch_r, w3, b3):
    c = w3.shape[1]
    out, _, _ = pl.pallas_call(
        _mm_pool_body,
        grid=(N_BLOCKS,),
        in_specs=[
            pl.BlockSpec((ROWS_BLK, D), lambda i: (i, 0)),
            pl.BlockSpec((NUM_CORES, ROWS_BLK, D), lambda i: (0, i, 0)),
            pl.BlockSpec((D, D), lambda i: (0, 0)),
            pl.BlockSpec((1, D), lambda i: (0, 0)),
            pl.BlockSpec((1, 1, ROWS_BLK), lambda i: (i, 0, 0)),
            pl.BlockSpec((D, c), lambda i: (0, 0)),
            pl.BlockSpec((1, c), lambda i: (0, 0)),
        ],
        out_specs=[
            pl.BlockSpec((G, c), lambda i: (0, 0)),
            pl.BlockSpec((G, D), lambda i: (0, 0)),
            pl.BlockSpec((G, D), lambda i: (0, 0)),
        ],
        out_shape=[
            jax.ShapeDtypeStruct((G, c), jnp.float32),
            jax.ShapeDtypeStruct((G, D), jnp.float32),
            jax.ShapeDtypeStruct((G, D), jnp.float32),
        ],
    )(h1, parts, w2, b2, batch_r, w3, b3)
    return out


def kernel(x, edge_index, batch, W1, b1, W2, b2, W3, b3):
    src = edge_index[0]
    dst = edge_index[1]

    pad = E_PAD - E
    src_p = jnp.concatenate(
        [src, jnp.arange(pad, dtype=jnp.int32) * 64 % N])
    junk = N + jnp.arange(pad, dtype=jnp.int32) % (N_ACC - N)
    dst_p = jnp.concatenate([dst, junk])
    srcr = src_p.reshape(NUM_WORKERS, STEPS, CHUNK)
    dstr = dst_p.reshape(NUM_WORKERS, STEPS, CHUNK)
    zeros = jnp.zeros((STRIPE, D), jnp.float32)

    sc_agg = _make_sc_agg()
    parts1 = sc_agg(x, srcr, dstr, zeros)
    h1 = _mm_relu(x, parts1, W1, b1.reshape(1, D))
    parts2 = sc_agg(h1, srcr, dstr, zeros)

    batch_r = batch.reshape(N_BLOCKS, 1, ROWS_BLK)
    out = _mm_pool(h1, parts2, W2, b2.reshape(1, D), batch_r,
                   W3, b3.reshape(1, -1))
    return out


# speedup vs baseline: 4.0463x; 1.4268x over previous
import functools

import jax
import jax.numpy as jnp
from jax import lax
from jax.experimental import pallas as pl
from jax.experimental.pallas import tpu as pltpu
from jax.experimental.pallas import tpu_sc as plsc

N = 10000
E = 320000
D = 128
G = 64

NUM_CORES = 2
NUM_SUBCORES = 16
NUM_WORKERS = NUM_CORES * NUM_SUBCORES
CHUNK = 128
NBUF = 2
IRING = 4
STEPS = 80
MROUNDS = STEPS // IRING
EDGES_PER_WORKER = STEPS * CHUNK
E_PAD = NUM_WORKERS * EDGES_PER_WORKER
N_ACC = 10112
STRIPE = N_ACC // NUM_SUBCORES

ROWS_BLK = 400
N_BLOCKS = N // ROWS_BLK


@functools.lru_cache(maxsize=1)
def _make_sc_agg():
    mesh = plsc.VectorSubcoreMesh(core_axis_name="c", subcore_axis_name="s",
                                  num_cores=NUM_CORES, num_subcores=NUM_SUBCORES)

    @functools.partial(
        pl.kernel,
        mesh=mesh,
        out_type=jax.ShapeDtypeStruct((NUM_CORES, N_ACC, D), jnp.float32),
        scratch_types=[
            pltpu.VMEM((IRING, 8, CHUNK), jnp.int32),
            pltpu.VMEM((NBUF, CHUNK, D), jnp.float32),
            pltpu.VMEM_SHARED((N_ACC, D), jnp.float32),
        ] + [pltpu.SemaphoreType.DMA] * (2 * NBUF + IRING),
    )
    def agg(feat, idxr, zeros, out, islot, rows, acc, *sems):
        semg = sems[:NBUF]
        semsc = sems[NBUF:2 * NBUF]
        semi = sems[2 * NBUF:]
        cid = lax.axis_index("c")
        sid = lax.axis_index("s")
        wid = sid * NUM_CORES + cid

        pltpu.sync_copy(zeros, acc.at[pl.ds(sid * STRIPE, STRIPE)])
        plsc.subcore_barrier()

        def iissue(c, s):
            pltpu.async_copy(idxr.at[wid, c], islot.at[s], semi[s])

        def iwait(s):
            pltpu.make_async_copy(idxr.at[wid, 0], islot.at[s], semi[s]).wait()

        def gissue(s, b):
            pltpu.async_copy(feat.at[islot.at[s].at[0]], rows.at[b], semg[b])

        def gwait(b):
            pltpu.make_async_copy(feat.at[islot.at[0].at[0]], rows.at[b],
                                  semg[b]).wait()

        def scat(s, b):
            pltpu.async_copy(rows.at[b], acc.at[islot.at[s].at[1]], semsc[b],
                             add=True)

        def swait(b):
            pltpu.make_async_copy(rows.at[b], acc.at[islot.at[0].at[1]],
                                  semsc[b]).wait()

        for s in range(IRING):
            iissue(s, s)
        for b in range(NBUF):
            iwait(b)
            gissue(b, b)

        def macro_round(m, carry):
            for k in range(IRING):
                b = k % NBUF
                c = m * IRING + k
                gwait(b)
                scat(k, b)
                swait(b)
                iissue(c + IRING, k)
                nxt = (k + NBUF) % IRING
                iwait(nxt)
                gissue(nxt, b)
            return carry

        lax.fori_loop(0, MROUNDS - 1, macro_round, 0)

        for k in range(NBUF):
            b = k % NBUF
            gwait(b)
            scat(k, b)
            swait(b)
            nxt = (k + NBUF) % IRING
            iwait(nxt)
            gissue(nxt, b)
        for k in range(NBUF, IRING):
            b = k % NBUF
            gwait(b)
            scat(k, b)
            swait(b)
        plsc.subcore_barrier()

        pltpu.sync_copy(
            acc.at[pl.ds(sid * STRIPE, STRIPE)],
            out.at[cid, pl.ds(sid * STRIPE, STRIPE)],
        )

    return agg


def _mm_relu_body(x_ref, p_ref, w_ref, b_ref, o_ref):
    xa = x_ref[...] + p_ref[0] + p_ref[1]
    h = jnp.dot(xa, w_ref[...], preferred_element_type=jnp.float32)
    o_ref[...] = jnp.maximum(h + b_ref[...], 0.0)


def _mm_relu(x, parts, w, b):
    return pl.pallas_call(
        _mm_relu_body,
        grid=(N_BLOCKS,),
        in_specs=[
            pl.BlockSpec((ROWS_BLK, D), lambda i: (i, 0)),
            pl.BlockSpec((NUM_CORES, ROWS_BLK, D), lambda i: (0, i, 0)),
            pl.BlockSpec((D, D), lambda i: (0, 0)),
            pl.BlockSpec((1, D), lambda i: (0, 0)),
        ],
        out_specs=pl.BlockSpec((ROWS_BLK, D), lambda i: (i, 0)),
        out_shape=jax.ShapeDtypeStruct((N, D), jnp.float32),
    )(x, parts, w, b)


def _mm_pool_body(x_ref, p_ref, w_ref, b_ref, batch_ref, w3_ref, b3_ref,
                  out_ref, sums_ref, cnts_ref):
    i = pl.program_id(0)
    xa = x_ref[...] + p_ref[0] + p_ref[1]
    h = jnp.dot(xa, w_ref[...], preferred_element_type=jnp.float32)
    h = jnp.maximum(h + b_ref[...], 0.0)

    seg = batch_ref[0, 0]
    onehot = (seg[:, None] == lax.broadcasted_iota(jnp.int32, (ROWS_BLK, G), 1))
    onehot = onehot.astype(jnp.float32)
    psum = lax.dot_general(onehot, h, (((0,), (0,)), ((), ())),
                           preferred_element_type=jnp.float32)
    pcnt = lax.dot_general(onehot, jnp.ones((ROWS_BLK, D), jnp.float32),
                           (((0,), (0,)), ((), ())),
                           preferred_element_type=jnp.float32)

    @pl.when(i == 0)
    def _():
        sums_ref[...] = jnp.zeros_like(sums_ref)
        cnts_ref[...] = jnp.zeros_like(cnts_ref)

    sums_ref[...] += psum
    cnts_ref[...] += pcnt

    @pl.when(i == N_BLOCKS - 1)
    def _():
        pooled = sums_ref[...] / jnp.maximum(cnts_ref[...], 1.0)
        logits = jnp.dot(pooled, w3_ref[...], preferred_element_type=jnp.float32)
        out_ref[...] = logits + b3_ref[...]


def _mm_pool(h1, parts, w2, b2, batch_r, w3, b3):
    c = w3.shape[1]
    out, _, _ = pl.pallas_call(
        _mm_pool_body,
        grid=(N_BLOCKS,),
        in_specs=[
            pl.BlockSpec((ROWS_BLK, D), lambda i: (i, 0)),
            pl.BlockSpec((NUM_CORES, ROWS_BLK, D), lambda i: (0, i, 0)),
            pl.BlockSpec((D, D), lambda i: (0, 0)),
            pl.BlockSpec((1, D), lambda i: (0, 0)),
            pl.BlockSpec((1, 1, ROWS_BLK), lambda i: (i, 0, 0)),
            pl.BlockSpec((D, c), lambda i: (0, 0)),
            pl.BlockSpec((1, c), lambda i: (0, 0)),
        ],
        out_specs=[
            pl.BlockSpec((G, c), lambda i: (0, 0)),
            pl.BlockSpec((G, D), lambda i: (0, 0)),
            pl.BlockSpec((G, D), lambda i: (0, 0)),
        ],
        out_shape=[
            jax.ShapeDtypeStruct((G, c), jnp.float32),
            jax.ShapeDtypeStruct((G, D), jnp.float32),
            jax.ShapeDtypeStruct((G, D), jnp.float32),
        ],
    )(h1, parts, w2, b2, batch_r, w3, b3)
    return out


def kernel(x, edge_index, batch, W1, b1, W2, b2, W3, b3):
    src = edge_index[0]
    dst = edge_index[1]

    pad = E_PAD - E
    src_p = jnp.concatenate(
        [src, jnp.arange(pad, dtype=jnp.int32) * 64 % N])
    junk = N + jnp.arange(pad, dtype=jnp.int32) % (N_ACC - N)
    dst_p = jnp.concatenate([dst, junk])
    srcr = src_p.reshape(NUM_WORKERS, STEPS, 1, CHUNK)
    dstr = dst_p.reshape(NUM_WORKERS, STEPS, 1, CHUNK)
    idxr = jnp.concatenate(
        [srcr, dstr,
         jnp.zeros((NUM_WORKERS, STEPS, 6, CHUNK), jnp.int32)], axis=2)
    zeros = jnp.zeros((STRIPE, D), jnp.float32)

    sc_agg = _make_sc_agg()
    parts1 = sc_agg(x, idxr, zeros)
    h1 = _mm_relu(x, parts1, W1, b1.reshape(1, D))
    parts2 = sc_agg(h1, idxr, zeros)

    batch_r = batch.reshape(N_BLOCKS, 1, ROWS_BLK)
    out = _mm_pool(h1, parts2, W2, b2.reshape(1, D), batch_r,
                   W3, b3.reshape(1, -1))
    return out


# speedup vs baseline: 4.1530x; 1.0264x over previous
import functools

import jax
import jax.numpy as jnp
from jax import lax
from jax.experimental import pallas as pl
from jax.experimental.pallas import tpu as pltpu
from jax.experimental.pallas import tpu_sc as plsc

N = 10000
E = 320000
D = 128
G = 64

NUM_CORES = 2
NUM_SUBCORES = 16
NUM_WORKERS = NUM_CORES * NUM_SUBCORES
CHUNK = 128
NBUF = 2
IRING = 4
STEPS = 80
MROUNDS = STEPS // IRING
EDGES_PER_WORKER = STEPS * CHUNK
E_PAD = NUM_WORKERS * EDGES_PER_WORKER
N_ACC = 10112
STRIPE = N_ACC // NUM_SUBCORES

ROWS_BLK = 400
N_BLOCKS = N // ROWS_BLK


@functools.lru_cache(maxsize=1)
def _make_sc_agg():
    mesh = plsc.VectorSubcoreMesh(core_axis_name="c", subcore_axis_name="s",
                                  num_cores=NUM_CORES, num_subcores=NUM_SUBCORES)

    @functools.partial(
        pl.kernel,
        mesh=mesh,
        out_type=jax.ShapeDtypeStruct((NUM_CORES, N_ACC, D), jnp.float32),
        scratch_types=[
            pltpu.VMEM((IRING, 2, CHUNK), jnp.int32),
            pltpu.VMEM((NBUF, CHUNK, D), jnp.float32),
            pltpu.VMEM_SHARED((N_ACC, D), jnp.float32),
        ] + [pltpu.SemaphoreType.DMA] * (2 * NBUF + IRING),
    )
    def agg(feat, idxr, zeros, out, islot, rows, acc, *sems):
        semg = sems[:NBUF]
        semsc = sems[NBUF:2 * NBUF]
        semi = sems[2 * NBUF:]
        cid = lax.axis_index("c")
        sid = lax.axis_index("s")
        wid = sid * NUM_CORES + cid

        pltpu.sync_copy(zeros, acc.at[pl.ds(sid * STRIPE, STRIPE)])
        plsc.subcore_barrier()

        def iissue(c, s):
            pltpu.async_copy(idxr.at[wid, c], islot.at[s], semi[s])

        def iwait(s):
            pltpu.make_async_copy(idxr.at[wid, 0], islot.at[s], semi[s]).wait()

        def gissue(s, b):
            pltpu.async_copy(feat.at[islot.at[s].at[0]], rows.at[b], semg[b])

        def gwait(b):
            pltpu.make_async_copy(feat.at[islot.at[0].at[0]], rows.at[b],
                                  semg[b]).wait()

        def scat(s, b):
            pltpu.async_copy(rows.at[b], acc.at[islot.at[s].at[1]], semsc[b],
                             add=True)

        def swait(b):
            pltpu.make_async_copy(rows.at[b], acc.at[islot.at[0].at[1]],
                                  semsc[b]).wait()

        for s in range(IRING):
            iissue(s, s)
        for b in range(NBUF):
            iwait(b)
            gissue(b, b)

        def macro_round(m, carry):
            for k in range(IRING):
                b = k % NBUF
                c = m * IRING + k
                gwait(b)
                scat(k, b)
                swait(b)
                iissue(c + IRING, k)
                nxt = (k + NBUF) % IRING
                iwait(nxt)
                gissue(nxt, b)
            return carry

        lax.fori_loop(0, MROUNDS - 1, macro_round, 0)

        for k in range(NBUF):
            b = k % NBUF
            gwait(b)
            scat(k, b)
            swait(b)
            nxt = (k + NBUF) % IRING
            iwait(nxt)
            gissue(nxt, b)
        for k in range(NBUF, IRING):
            b = k % NBUF
            gwait(b)
            scat(k, b)
            swait(b)
        plsc.subcore_barrier()

        pltpu.sync_copy(
            acc.at[pl.ds(sid * STRIPE, STRIPE)],
            out.at[cid, pl.ds(sid * STRIPE, STRIPE)],
        )

    return agg


def _mm_relu_body(x_ref, p_ref, w_ref, b_ref, o_ref):
    xa = x_ref[...] + p_ref[0] + p_ref[1]
    h = jnp.dot(xa, w_ref[...], preferred_element_type=jnp.float32)
    o_ref[...] = jnp.maximum(h + b_ref[...], 0.0)


def _mm_relu(x, parts, w, b):
    return pl.pallas_call(
        _mm_relu_body,
        grid=(N_BLOCKS,),
        in_specs=[
            pl.BlockSpec((ROWS_BLK, D), lambda i: (i, 0)),
            pl.BlockSpec((NUM_CORES, ROWS_BLK, D), lambda i: (0, i, 0)),
            pl.BlockSpec((D, D), lambda i: (0, 0)),
            pl.BlockSpec((1, D), lambda i: (0, 0)),
        ],
        out_specs=pl.BlockSpec((ROWS_BLK, D), lambda i: (i, 0)),
        out_shape=jax.ShapeDtypeStruct((N, D), jnp.float32),
    )(x, parts, w, b)


def _mm_pool_body(x_ref, p_ref, w_ref, b_ref, batch_ref, w3_ref, b3_ref,
                  out_ref, sums_ref, cnts_ref):
    i = pl.program_id(0)
    xa = x_ref[...] + p_ref[0] + p_ref[1]
    h = jnp.dot(xa, w_ref[...], preferred_element_type=jnp.float32)
    h = jnp.maximum(h + b_ref[...], 0.0)

    seg = batch_ref[0, 0]
    onehot = (seg[:, None] == lax.broadcasted_iota(jnp.int32, (ROWS_BLK, G), 1))
    onehot = onehot.astype(jnp.float32)
    psum = lax.dot_general(onehot, h, (((0,), (0,)), ((), ())),
                           preferred_element_type=jnp.float32)
    pcnt = lax.dot_general(onehot, jnp.ones((ROWS_BLK, D), jnp.float32),
                           (((0,), (0,)), ((), ())),
                           preferred_element_type=jnp.float32)

    @pl.when(i == 0)
    def _():
        sums_ref[...] = jnp.zeros_like(sums_ref)
        cnts_ref[...] = jnp.zeros_like(cnts_ref)

    sums_ref[...] += psum
    cnts_ref[...] += pcnt

    @pl.when(i == N_BLOCKS - 1)
    def _():
        pooled = sums_ref[...] / jnp.maximum(cnts_ref[...], 1.0)
        logits = jnp.dot(pooled, w3_ref[...], preferred_element_type=jnp.float32)
        out_ref[...] = logits + b3_ref[...]


def _mm_pool(h1, parts, w2, b2, batch_r, w3, b3):
    c = w3.shape[1]
    out, _, _ = pl.pallas_call(
        _mm_pool_body,
        grid=(N_BLOCKS,),
        in_specs=[
            pl.BlockSpec((ROWS_BLK, D), lambda i: (i, 0)),
            pl.BlockSpec((NUM_CORES, ROWS_BLK, D), lambda i: (0, i, 0)),
            pl.BlockSpec((D, D), lambda i: (0, 0)),
            pl.BlockSpec((1, D), lambda i: (0, 0)),
            pl.BlockSpec((1, 1, ROWS_BLK), lambda i: (i, 0, 0)),
            pl.BlockSpec((D, c), lambda i: (0, 0)),
            pl.BlockSpec((1, c), lambda i: (0, 0)),
        ],
        out_specs=[
            pl.BlockSpec((G, c), lambda i: (0, 0)),
            pl.BlockSpec((G, D), lambda i: (0, 0)),
            pl.BlockSpec((G, D), lambda i: (0, 0)),
        ],
        out_shape=[
            jax.ShapeDtypeStruct((G, c), jnp.float32),
            jax.ShapeDtypeStruct((G, D), jnp.float32),
            jax.ShapeDtypeStruct((G, D), jnp.float32),
        ],
    )(h1, parts, w2, b2, batch_r, w3, b3)
    return out


def kernel(x, edge_index, batch, W1, b1, W2, b2, W3, b3):
    src = edge_index[0]
    dst = edge_index[1]

    pad = E_PAD - E
    src_p = jnp.concatenate(
        [src, jnp.arange(pad, dtype=jnp.int32) * 64 % N])
    junk = N + jnp.arange(pad, dtype=jnp.int32) % (N_ACC - N)
    dst_p = jnp.concatenate([dst, junk])
    srcr = src_p.reshape(NUM_WORKERS, STEPS, 1, CHUNK)
    dstr = dst_p.reshape(NUM_WORKERS, STEPS, 1, CHUNK)
    idxr = jnp.concatenate([srcr, dstr], axis=2)
    zeros = jnp.zeros((STRIPE, D), jnp.float32)

    sc_agg = _make_sc_agg()
    parts1 = sc_agg(x, idxr, zeros)
    h1 = _mm_relu(x, parts1, W1, b1.reshape(1, D))
    parts2 = sc_agg(h1, idxr, zeros)

    batch_r = batch.reshape(N_BLOCKS, 1, ROWS_BLK)
    out = _mm_pool(h1, parts2, W2, b2.reshape(1, D), batch_r,
                   W3, b3.reshape(1, -1))
    return out


# speedup vs baseline: 4.1547x; 1.0004x over previous
import functools

import jax
import jax.numpy as jnp
from jax import lax
from jax.experimental import pallas as pl
from jax.experimental.pallas import tpu as pltpu
from jax.experimental.pallas import tpu_sc as plsc

N = 10000
E = 320000
D = 128
G = 64

NUM_CORES = 2
NUM_SUBCORES = 16
NUM_WORKERS = NUM_CORES * NUM_SUBCORES
CHUNK = 128
NBUF = 2
IRING = 8
STEPS = 80
MROUNDS = STEPS // IRING
EDGES_PER_WORKER = STEPS * CHUNK
E_PAD = NUM_WORKERS * EDGES_PER_WORKER
N_ACC = 10112
STRIPE = N_ACC // NUM_SUBCORES

ROWS_BLK = 400
N_BLOCKS = N // ROWS_BLK


@functools.lru_cache(maxsize=1)
def _make_sc_agg():
    mesh = plsc.VectorSubcoreMesh(core_axis_name="c", subcore_axis_name="s",
                                  num_cores=NUM_CORES, num_subcores=NUM_SUBCORES)

    @functools.partial(
        pl.kernel,
        mesh=mesh,
        out_type=jax.ShapeDtypeStruct((NUM_CORES, N_ACC, D), jnp.float32),
        scratch_types=[
            pltpu.VMEM((IRING, 2, CHUNK), jnp.int32),
            pltpu.VMEM((NBUF, CHUNK, D), jnp.float32),
            pltpu.VMEM_SHARED((N_ACC, D), jnp.float32),
        ] + [pltpu.SemaphoreType.DMA] * (2 * NBUF + IRING),
    )
    def agg(feat, idxr, zeros, out, islot, rows, acc, *sems):
        semg = sems[:NBUF]
        semsc = sems[NBUF:2 * NBUF]
        semi = sems[2 * NBUF:]
        cid = lax.axis_index("c")
        sid = lax.axis_index("s")
        wid = sid * NUM_CORES + cid

        pltpu.sync_copy(zeros, acc.at[pl.ds(sid * STRIPE, STRIPE)])
        plsc.subcore_barrier()

        def iissue(c, s):
            pltpu.async_copy(idxr.at[wid, c], islot.at[s], semi[s])

        def iwait(s):
            pltpu.make_async_copy(idxr.at[wid, 0], islot.at[s], semi[s]).wait()

        def gissue(s, b):
            pltpu.async_copy(feat.at[islot.at[s].at[0]], rows.at[b], semg[b])

        def gwait(b):
            pltpu.make_async_copy(feat.at[islot.at[0].at[0]], rows.at[b],
                                  semg[b]).wait()

        def scat(s, b):
            pltpu.async_copy(rows.at[b], acc.at[islot.at[s].at[1]], semsc[b],
                             add=True)

        def swait(b):
            pltpu.make_async_copy(rows.at[b], acc.at[islot.at[0].at[1]],
                                  semsc[b]).wait()

        for s in range(IRING):
            iissue(s, s)
        for b in range(NBUF):
            iwait(b)
            gissue(b, b)

        def macro_round(m, carry):
            for k in range(IRING):
                b = k % NBUF
                c = m * IRING + k
                gwait(b)
                scat(k, b)
                nxt = (k + NBUF) % IRING
                iwait(nxt)
                swait(b)
                iissue(c + IRING, k)
                gissue(nxt, b)
            return carry

        lax.fori_loop(0, MROUNDS - 1, macro_round, 0)

        for k in range(IRING):
            b = k % NBUF
            gwait(b)
            scat(k, b)
            swait(b)
            if k < IRING - NBUF:
                nxt = (k + NBUF) % IRING
                iwait(nxt)
                gissue(nxt, b)
        plsc.subcore_barrier()

        pltpu.sync_copy(
            acc.at[pl.ds(sid * STRIPE, STRIPE)],
            out.at[cid, pl.ds(sid * STRIPE, STRIPE)],
        )

    return agg


def _mm_relu_body(x_ref, p_ref, w_ref, b_ref, o_ref):
    xa = x_ref[...] + p_ref[0] + p_ref[1]
    h = jnp.dot(xa, w_ref[...], preferred_element_type=jnp.float32)
    o_ref[...] = jnp.maximum(h + b_ref[...], 0.0)


def _mm_relu(x, parts, w, b):
    return pl.pallas_call(
        _mm_relu_body,
        grid=(N_BLOCKS,),
        in_specs=[
            pl.BlockSpec((ROWS_BLK, D), lambda i: (i, 0)),
            pl.BlockSpec((NUM_CORES, ROWS_BLK, D), lambda i: (0, i, 0)),
            pl.BlockSpec((D, D), lambda i: (0, 0)),
            pl.BlockSpec((1, D), lambda i: (0, 0)),
        ],
        out_specs=pl.BlockSpec((ROWS_BLK, D), lambda i: (i, 0)),
        out_shape=jax.ShapeDtypeStruct((N, D), jnp.float32),
    )(x, parts, w, b)


def _mm_pool_body(x_ref, p_ref, w_ref, b_ref, batch_ref, w3_ref, b3_ref,
                  out_ref, sums_ref, cnts_ref):
    i = pl.program_id(0)
    xa = x_ref[...] + p_ref[0] + p_ref[1]
    h = jnp.dot(xa, w_ref[...], preferred_element_type=jnp.float32)
    h = jnp.maximum(h + b_ref[...], 0.0)

    seg = batch_ref[0, 0]
    onehot = (seg[:, None] == lax.broadcasted_iota(jnp.int32, (ROWS_BLK, G), 1))
    onehot = onehot.astype(jnp.float32)
    psum = lax.dot_general(onehot, h, (((0,), (0,)), ((), ())),
                           preferred_element_type=jnp.float32)
    pcnt = lax.dot_general(onehot, jnp.ones((ROWS_BLK, D), jnp.float32),
                           (((0,), (0,)), ((), ())),
                           preferred_element_type=jnp.float32)

    @pl.when(i == 0)
    def _():
        sums_ref[...] = jnp.zeros_like(sums_ref)
        cnts_ref[...] = jnp.zeros_like(cnts_ref)

    sums_ref[...] += psum
    cnts_ref[...] += pcnt

    @pl.when(i == N_BLOCKS - 1)
    def _():
        pooled = sums_ref[...] / jnp.maximum(cnts_ref[...], 1.0)
        logits = jnp.dot(pooled, w3_ref[...], preferred_element_type=jnp.float32)
        out_ref[...] = logits + b3_ref[...]


def _mm_pool(h1, parts, w2, b2, batch_r, w3, b3):
    c = w3.shape[1]
    out, _, _ = pl.pallas_call(
        _mm_pool_body,
        grid=(N_BLOCKS,),
        in_specs=[
            pl.BlockSpec((ROWS_BLK, D), lambda i: (i, 0)),
            pl.BlockSpec((NUM_CORES, ROWS_BLK, D), lambda i: (0, i, 0)),
            pl.BlockSpec((D, D), lambda i: (0, 0)),
            pl.BlockSpec((1, D), lambda i: (0, 0)),
            pl.BlockSpec((1, 1, ROWS_BLK), lambda i: (i, 0, 0)),
            pl.BlockSpec((D, c), lambda i: (0, 0)),
            pl.BlockSpec((1, c), lambda i: (0, 0)),
        ],
        out_specs=[
            pl.BlockSpec((G, c), lambda i: (0, 0)),
            pl.BlockSpec((G, D), lambda i: (0, 0)),
            pl.BlockSpec((G, D), lambda i: (0, 0)),
        ],
        out_shape=[
            jax.ShapeDtypeStruct((G, c), jnp.float32),
            jax.ShapeDtypeStruct((G, D), jnp.float32),
            jax.ShapeDtypeStruct((G, D), jnp.float32),
        ],
    )(h1, parts, w2, b2, batch_r, w3, b3)
    return out


def kernel(x, edge_index, batch, W1, b1, W2, b2, W3, b3):
    src = edge_index[0]
    dst = edge_index[1]

    pad = E_PAD - E
    src_p = jnp.concatenate(
        [src, jnp.arange(pad, dtype=jnp.int32) * 64 % N])
    junk = N + jnp.arange(pad, dtype=jnp.int32) % (N_ACC - N)
    dst_p = jnp.concatenate([dst, junk])
    srcr = src_p.reshape(NUM_WORKERS, STEPS, 1, CHUNK)
    dstr = dst_p.reshape(NUM_WORKERS, STEPS, 1, CHUNK)
    idxr = jnp.concatenate([srcr, dstr], axis=2)
    zeros = jnp.zeros((STRIPE, D), jnp.float32)

    sc_agg = _make_sc_agg()
    parts1 = sc_agg(x, idxr, zeros)
    h1 = _mm_relu(x, parts1, W1, b1.reshape(1, D))
    parts2 = sc_agg(h1, idxr, zeros)

    batch_r = batch.reshape(N_BLOCKS, 1, ROWS_BLK)
    out = _mm_pool(h1, parts2, W2, b2.reshape(1, D), batch_r,
                   W3, b3.reshape(1, -1))
    return out


# speedup vs baseline: 4.4484x; 1.0707x over previous
import functools

import jax
import jax.numpy as jnp
from jax import lax
from jax.experimental import pallas as pl
from jax.experimental.pallas import tpu as pltpu
from jax.experimental.pallas import tpu_sc as plsc

N = 10000
E = 320000
D = 128
G = 64

NUM_CORES = 2
NUM_SUBCORES = 16
NUM_WORKERS = NUM_CORES * NUM_SUBCORES
CHUNK = 128
NBUF = 2
IRING = 8
STEPS = 80
MROUNDS = STEPS // IRING
EDGES_PER_WORKER = STEPS * CHUNK
E_PAD = NUM_WORKERS * EDGES_PER_WORKER
N_ACC = 10112
STRIPE = N_ACC // NUM_SUBCORES

ROWS_BLK = 2000
N_BLOCKS = N // ROWS_BLK


@functools.lru_cache(maxsize=1)
def _make_sc_agg():
    mesh = plsc.VectorSubcoreMesh(core_axis_name="c", subcore_axis_name="s",
                                  num_cores=NUM_CORES, num_subcores=NUM_SUBCORES)

    @functools.partial(
        pl.kernel,
        mesh=mesh,
        out_type=jax.ShapeDtypeStruct((NUM_CORES, N_ACC, D), jnp.float32),
        scratch_types=[
            pltpu.VMEM((IRING, 2, CHUNK), jnp.int32),
            pltpu.VMEM((NBUF, CHUNK, D), jnp.float32),
            pltpu.VMEM_SHARED((N_ACC, D), jnp.float32),
        ] + [pltpu.SemaphoreType.DMA] * (2 * NBUF + IRING),
    )
    def agg(feat, idxr, zeros, out, islot, rows, acc, *sems):
        semg = sems[:NBUF]
        semsc = sems[NBUF:2 * NBUF]
        semi = sems[2 * NBUF:]
        cid = lax.axis_index("c")
        sid = lax.axis_index("s")
        wid = sid * NUM_CORES + cid

        pltpu.sync_copy(zeros, acc.at[pl.ds(sid * STRIPE, STRIPE)])
        plsc.subcore_barrier()

        def iissue(c, s):
            pltpu.async_copy(idxr.at[wid, c], islot.at[s], semi[s])

        def iwait(s):
            pltpu.make_async_copy(idxr.at[wid, 0], islot.at[s], semi[s]).wait()

        def gissue(s, b):
            pltpu.async_copy(feat.at[islot.at[s].at[0]], rows.at[b], semg[b])

        def gwait(b):
            pltpu.make_async_copy(feat.at[islot.at[0].at[0]], rows.at[b],
                                  semg[b]).wait()

        def scat(s, b):
            pltpu.async_copy(rows.at[b], acc.at[islot.at[s].at[1]], semsc[b],
                             add=True)

        def swait(b):
            pltpu.make_async_copy(rows.at[b], acc.at[islot.at[0].at[1]],
                                  semsc[b]).wait()

        for s in range(IRING):
            iissue(s, s)
        for b in range(NBUF):
            iwait(b)
            gissue(b, b)

        def macro_round(m, carry):
            for k in range(IRING):
                b = k % NBUF
                c = m * IRING + k
                gwait(b)
                scat(k, b)
                nxt = (k + NBUF) % IRING
                iwait(nxt)
                swait(b)
                iissue(c + IRING, k)
                gissue(nxt, b)
            return carry

        lax.fori_loop(0, MROUNDS - 1, macro_round, 0)

        for k in range(IRING):
            b = k % NBUF
            gwait(b)
            scat(k, b)
            swait(b)
            if k < IRING - NBUF:
                nxt = (k + NBUF) % IRING
                iwait(nxt)
                gissue(nxt, b)
        plsc.subcore_barrier()

        pltpu.sync_copy(
            acc.at[pl.ds(sid * STRIPE, STRIPE)],
            out.at[cid, pl.ds(sid * STRIPE, STRIPE)],
        )

    return agg


def _mm_relu_body(x_ref, p_ref, w_ref, b_ref, o_ref):
    xa = x_ref[...] + p_ref[0] + p_ref[1]
    h = jnp.dot(xa, w_ref[...], preferred_element_type=jnp.float32)
    o_ref[...] = jnp.maximum(h + b_ref[...], 0.0)


def _mm_relu(x, parts, w, b):
    return pl.pallas_call(
        _mm_relu_body,
        grid=(N_BLOCKS,),
        in_specs=[
            pl.BlockSpec((ROWS_BLK, D), lambda i: (i, 0)),
            pl.BlockSpec((NUM_CORES, ROWS_BLK, D), lambda i: (0, i, 0)),
            pl.BlockSpec((D, D), lambda i: (0, 0)),
            pl.BlockSpec((1, D), lambda i: (0, 0)),
        ],
        out_specs=pl.BlockSpec((ROWS_BLK, D), lambda i: (i, 0)),
        out_shape=jax.ShapeDtypeStruct((N, D), jnp.float32),
    )(x, parts, w, b)


def _mm_pool_body(x_ref, p_ref, w_ref, b_ref, batch_ref, w3_ref, b3_ref,
                  out_ref, sums_ref, cnts_ref):
    i = pl.program_id(0)
    xa = x_ref[...] + p_ref[0] + p_ref[1]
    h = jnp.dot(xa, w_ref[...], preferred_element_type=jnp.float32)
    h = jnp.maximum(h + b_ref[...], 0.0)

    seg = batch_ref[0, 0]
    onehot = (seg[:, None] == lax.broadcasted_iota(jnp.int32, (ROWS_BLK, G), 1))
    onehot = onehot.astype(jnp.float32)
    psum = lax.dot_general(onehot, h, (((0,), (0,)), ((), ())),
                           preferred_element_type=jnp.float32)
    pcnt = lax.dot_general(onehot, jnp.ones((ROWS_BLK, D), jnp.float32),
                           (((0,), (0,)), ((), ())),
                           preferred_element_type=jnp.float32)

    @pl.when(i == 0)
    def _():
        sums_ref[...] = jnp.zeros_like(sums_ref)
        cnts_ref[...] = jnp.zeros_like(cnts_ref)

    sums_ref[...] += psum
    cnts_ref[...] += pcnt

    @pl.when(i == N_BLOCKS - 1)
    def _():
        pooled = sums_ref[...] / jnp.maximum(cnts_ref[...], 1.0)
        logits = jnp.dot(pooled, w3_ref[...], preferred_element_type=jnp.float32)
        out_ref[...] = logits + b3_ref[...]


def _mm_pool(h1, parts, w2, b2, batch_r, w3, b3):
    c = w3.shape[1]
    out, _, _ = pl.pallas_call(
        _mm_pool_body,
        grid=(N_BLOCKS,),
        in_specs=[
            pl.BlockSpec((ROWS_BLK, D), lambda i: (i, 0)),
            pl.BlockSpec((NUM_CORES, ROWS_BLK, D), lambda i: (0, i, 0)),
            pl.BlockSpec((D, D), lambda i: (0, 0)),
            pl.BlockSpec((1, D), lambda i: (0, 0)),
            pl.BlockSpec((1, 1, ROWS_BLK), lambda i: (i, 0, 0)),
            pl.BlockSpec((D, c), lambda i: (0, 0)),
            pl.BlockSpec((1, c), lambda i: (0, 0)),
        ],
        out_specs=[
            pl.BlockSpec((G, c), lambda i: (0, 0)),
            pl.BlockSpec((G, D), lambda i: (0, 0)),
            pl.BlockSpec((G, D), lambda i: (0, 0)),
        ],
        out_shape=[
            jax.ShapeDtypeStruct((G, c), jnp.float32),
            jax.ShapeDtypeStruct((G, D), jnp.float32),
            jax.ShapeDtypeStruct((G, D), jnp.float32),
        ],
    )(h1, parts, w2, b2, batch_r, w3, b3)
    return out


def kernel(x, edge_index, batch, W1, b1, W2, b2, W3, b3):
    src = edge_index[0]
    dst = edge_index[1]

    pad = E_PAD - E
    src_p = jnp.concatenate(
        [src, jnp.arange(pad, dtype=jnp.int32) * 64 % N])
    junk = N + jnp.arange(pad, dtype=jnp.int32) % (N_ACC - N)
    dst_p = jnp.concatenate([dst, junk])
    srcr = src_p.reshape(NUM_WORKERS, STEPS, 1, CHUNK)
    dstr = dst_p.reshape(NUM_WORKERS, STEPS, 1, CHUNK)
    idxr = jnp.concatenate([srcr, dstr], axis=2)
    zeros = jnp.zeros((STRIPE, D), jnp.float32)

    sc_agg = _make_sc_agg()
    parts1 = sc_agg(x, idxr, zeros)
    h1 = _mm_relu(x, parts1, W1, b1.reshape(1, D))
    parts2 = sc_agg(h1, idxr, zeros)

    batch_r = batch.reshape(N_BLOCKS, 1, ROWS_BLK)
    out = _mm_pool(h1, parts2, W2, b2.reshape(1, D), batch_r,
                   W3, b3.reshape(1, -1))
    return out
